# baseline (speedup 1.0000x reference)
import functools

import jax
import jax.numpy as jnp
import numpy as np
from jax import lax
from jax.experimental import pallas as pl
from jax.experimental.pallas import tpu as pltpu
from jax.experimental.pallas import tpu_sc as plsc

BATCH = 4
N_NODES = 50000
M_NODES = 12500
SEQ = 9
CH = 128
NNZ = 37500

BN = 400


def _mm_body(x_ref, w_ref, y_ref):
    xb = x_ref[...].reshape(BATCH * BN, CH)
    y = jnp.dot(xb, w_ref[...], preferred_element_type=jnp.float32)
    yb = y.astype(jnp.bfloat16)
    for s in range(SEQ):
        for p in range(BATCH // 2):
            lo = yb[(2 * p) * BN:(2 * p + 1) * BN, s * CH:(s + 1) * CH]
            hi = yb[(2 * p + 1) * BN:(2 * p + 2) * BN, s * CH:(s + 1) * CH]
            lo32 = jax.lax.bitcast_convert_type(
                lo, jnp.uint16).astype(jnp.int32)
            hi32 = jax.lax.bitcast_convert_type(
                hi, jnp.uint16).astype(jnp.int32)
            y_ref[s * 2 + p] = jnp.bitwise_or(
                lo32, jnp.left_shift(hi32, 16))


def _stage1(x_bf, w_all):
    return pl.pallas_call(
        _mm_body,
        grid=(N_NODES // BN,),
        in_specs=[
            pl.BlockSpec((BATCH, BN, CH), lambda i: (0, i, 0)),
            pl.BlockSpec((CH, SEQ * CH), lambda i: (0, 0)),
        ],
        out_specs=pl.BlockSpec((SEQ * 2, BN, CH), lambda i: (0, i, 0)),
        out_shape=jax.ShapeDtypeStruct((SEQ * 2, N_NODES, CH), jnp.int32),
    )(x_bf, w_all)


CN = 16
NCHUNK = N_NODES // CN
NWORK = 32
KPW = (NCHUNK + NWORK - 1) // NWORK
NROW = SEQ * 2 * CN
IDXW = NROW
_STREAMS = ((0, 128), (128, 128), (256, 32))

@functools.lru_cache(maxsize=None)
def _mesh():
    return plsc.VectorSubcoreMesh(core_axis_name="c", subcore_axis_name="s")


@functools.lru_cache(maxsize=None)
def _stage2_kernel():
    return pl.kernel(
        _stage2_body,
        out_type=jax.ShapeDtypeStruct((BATCH, N_NODES, CH), jnp.float32),
        mesh=_mesh(),
        compiler_params=pltpu.CompilerParams(needs_layout_passes=False),
        scratch_types=[
            pltpu.VMEM((KPW * IDXW,), jnp.int32),
            pltpu.VMEM((2, NROW, CH), jnp.int32),
            pltpu.VMEM((2, BATCH, CN, CH), jnp.float32),
            pltpu.VMEM((CH,), jnp.float32),
            pltpu.SemaphoreType.DMA,
            pltpu.SemaphoreType.DMA,
            pltpu.SemaphoreType.DMA,
            pltpu.SemaphoreType.DMA,
        ],
    )


def _bf16_pair_to_f32(tot):
    return plsc.unpack(tot, format=plsc.PackFormat.INTERLEAVED)


def _elu(v):
    return jnp.where(v > 0, v, jnp.exp(v) - 1.0)


def _stage2_body(y_hbm, idxc_hbm, bias_hbm, o_hbm, idx_v, buf_v,
                 res_v, bias_v, sem0, sem1, osem0, osem1):
    wid = lax.axis_index("s") * 2 + lax.axis_index("c")
    sems = (sem0, sem1)
    osems = (osem0, osem1)
    pltpu.sync_copy(bias_hbm, bias_v)
    pltpu.sync_copy(idxc_hbm.at[pl.ds(wid * (KPW * IDXW), KPW * IDXW)], idx_v)

    def fire(j, par):
        for off, n in _STREAMS:
            pltpu.async_copy(
                y_hbm.at[idx_v.at[pl.ds(j * IDXW + off, n)]],
                buf_v.at[par, pl.ds(off, n)], sems[par])

    def drain(j, par):
        for off, n in _STREAMS:
            pltpu.make_async_copy(
                y_hbm.at[idx_v.at[pl.ds(j * IDXW + off, n)]],
                buf_v.at[par, pl.ds(off, n)], sems[par]).wait()

    def drain_out(par):
        for b in range(BATCH):
            pltpu.make_async_copy(
                res_v.at[par, b], o_hbm.at[b, pl.ds(0, CN)],
                osems[par]).wait()

    def consume(ck, j, par):
        drain(j, par)

        @pl.when(j >= 2)
        def _():
            drain_out(par)

        def row_body(r, c):
            for p in range(2):
                for g in range(CH // 16):
                    sl16 = pl.ds(g * 16, 16)
                    terms = [
                        plsc.bitcast(
                            buf_v[par, (s * 2 + p) * CN + r, sl16],
                            jnp.bfloat16)
                        for s in range(SEQ)
                    ]
                    while len(terms) > 1:
                        nxt = []
                        for t in range(0, len(terms) - 1, 2):
                            nxt.append(terms[t] + terms[t + 1])
                        if len(terms) % 2:
                            nxt.append(terms[-1])
                        terms = nxt
                    lo, hi = _bf16_pair_to_f32(terms[0])
                    bias = bias_v[sl16]
                    res_v[par, 2 * p, r, sl16] = _elu(lo + bias)
                    res_v[par, 2 * p + 1, r, sl16] = _elu(hi + bias)
            return c

        lax.fori_loop(0, CN, row_body, 0)
        base = ck * CN
        for b in range(BATCH):
            pltpu.async_copy(
                res_v.at[par, b], o_hbm.at[b, pl.ds(base, CN)], osems[par])

    fire(0, 0)

    def pair_body(jj, carry):
        for par in range(2):
            j = 2 * jj + par
            ck = wid + j * NWORK
            nck = ck + NWORK

            @pl.when(nck < NCHUNK)
            def _():
                fire(j + 1, 1 - par)

            @pl.when(ck < NCHUNK)
            def _():
                consume(ck, j, par)

        return carry

    lax.fori_loop(0, (KPW + 1) // 2, pair_body, 0)
    drain_out(0)
    drain_out(1)


EC = 128
NEC = (NNZ + EC - 1) // EC
EP = NEC * EC
MR = 24
M_PAD = 12544
RZ = 128
NRC = M_PAD // RZ


@functools.lru_cache(maxsize=None)
def _stage3_kernel():
    return pl.kernel(
        _stage3_body,
        out_type=jax.ShapeDtypeStruct((BATCH * M_PAD, CH), jnp.float32),
        mesh=_mesh(),
        compiler_params=pltpu.CompilerParams(needs_layout_passes=False),
        scratch_types=[
            pltpu.VMEM((2, MR, CH), jnp.int32),
            pltpu.VMEM((EC, CH), jnp.float32),
            pltpu.VMEM_SHARED((M_PAD, CH), jnp.float32),
            pltpu.SemaphoreType.DMA,
            pltpu.SemaphoreType.DMA,
            pltpu.SemaphoreType.DMA,
        ],
    )


def _stage3_body(o_hbm, emeta_hbm, p_hbm, ebuf_v, g_v, shared,
                 semm0, semm1, sem):
    cid = lax.axis_index("c")
    sid = lax.axis_index("s")
    semms = (semm0, semm1)
    nech = (NEC + 15) // 16

    def meta_fire(i, par):
        ck = sid + i * 16
        pltpu.async_copy(emeta_hbm.at[pl.ds(ck * MR, MR)],
                         ebuf_v.at[par], semms[par])

    def meta_wait(i, par):
        ck = sid + i * 16
        pltpu.make_async_copy(emeta_hbm.at[pl.ds(ck * MR, MR)],
                              ebuf_v.at[par], semms[par]).wait()

    for bp in range(2):
        b = bp * 2 + cid

        def zb(r, c):
            for j in range(CH // 16):
                g_v[r, pl.ds(j * 16, 16)] = jnp.zeros((16,), jnp.float32)
            return c

        lax.fori_loop(0, RZ, zb, 0)

        def zchunk(i, c):
            ck = sid + i * 16

            @pl.when(ck < NRC)
            def _():
                pltpu.sync_copy(g_v, shared.at[pl.ds(ck * RZ, RZ)])

            return c

        lax.fori_loop(0, (NRC + 15) // 16, zchunk, 0)
        plsc.subcore_barrier()

        meta_fire(0, 0)

        def epair(ii, c):
            for par in range(2):
                i = 2 * ii + par
                ck = sid + i * 16

                @pl.when(sid + (i + 1) * 16 < NEC)
                def _():
                    meta_fire(i + 1, 1 - par)

                @pl.when(ck < NEC)
                def _():
                    meta_wait(i, par)
                    off = b * N_NODES
                    for j in range(CH // 16):
                        sl = pl.ds(j * 16, 16)
                        ebuf_v[par, 0, sl] = ebuf_v[par, 0, sl] + off
                    pltpu.async_copy(
                        o_hbm.at[ebuf_v.at[par, 0]], g_v, sem).wait()

                    def scale(r, cc):
                        for cg in range(8):
                            vk = plsc.bitcast(
                                ebuf_v[par, 2 + r, pl.ds(cg * 16, 16)],
                                jnp.float32)
                            e = r * 8 + cg
                            for j in range(CH // 16):
                                sl = (e, pl.ds(j * 16, 16))
                                g_v[sl] = g_v[sl] * vk
                        return cc

                    lax.fori_loop(0, 16, scale, 0)
                    pltpu.sync_copy(g_v, shared.at[ebuf_v.at[par, 1]],
                                    add=True)

            return c

        lax.fori_loop(0, (nech + 1) // 2, epair, 0)
        plsc.subcore_barrier()

        def wchunk(i, c):
            ck = sid + i * 16

            @pl.when(ck < NRC)
            def _():
                r0 = ck * RZ
                pltpu.sync_copy(shared.at[pl.ds(r0, RZ)],
                                p_hbm.at[pl.ds(b * M_PAD + r0, RZ)])

            return c

        lax.fori_loop(0, (NRC + 15) // 16, wchunk, 0)
        plsc.subcore_barrier()


def kernel(x, W, b, down_value, indices, down_row, down_col):
    w_all = W.reshape(SEQ, CH, CH).transpose(1, 0, 2).reshape(CH, SEQ * CH)
    y = _stage1(x.astype(jnp.bfloat16), w_all.astype(jnp.bfloat16))

    idxt = indices.astype(jnp.int32).T.reshape(SEQ, NCHUNK, CN)
    offs = ((jnp.arange(SEQ) * 2)[:, None, None, None]
            + jnp.arange(2)[None, None, :, None]) * N_NODES
    gidx = idxt[:, :, None, :] + offs
    idxc = gidx.transpose(1, 0, 2, 3).reshape(NCHUNK, IDXW)
    idxc = jnp.pad(idxc, ((0, KPW * NWORK - NCHUNK), (0, 0)))
    idxc = idxc.reshape(KPW, NWORK, IDXW).transpose(1, 0, 2).reshape(-1)
    o = _stage2_kernel()(y.reshape(SEQ * 2 * N_NODES, CH), idxc, b)

    pad = EP - NNZ
    colp = jnp.pad(down_col.astype(jnp.int32), (0, pad)).reshape(NEC, 1, CH)
    rowp = jnp.pad(down_row.astype(jnp.int32), (0, pad)).reshape(NEC, 1, CH)
    val16 = jax.lax.bitcast_convert_type(
        jnp.repeat(jnp.pad(down_value, (0, pad)), 16),
        jnp.int32).reshape(NEC, 16, CH)
    zpad = jnp.zeros((NEC, MR - 18, CH), jnp.int32)
    emeta = jnp.concatenate([colp, rowp, val16, zpad], axis=1)
    o2 = o.reshape(BATCH * N_NODES, CH)
    p = _stage3_kernel()(o2, emeta.reshape(NEC * MR, CH))
    return p.reshape(BATCH, M_PAD, CH)[:, :M_NODES]


# speedup vs baseline: 1.0462x; 1.0462x over previous
import functools

import jax
import jax.numpy as jnp
import numpy as np
from jax import lax
from jax.experimental import pallas as pl
from jax.experimental.pallas import tpu as pltpu
from jax.experimental.pallas import tpu_sc as plsc

BATCH = 4
N_NODES = 50000
M_NODES = 12500
SEQ = 9
CH = 128
NNZ = 37500

BN = 1000


def _mm_body(x_ref, w_ref, y_ref):
    xb = x_ref[...].reshape(BATCH * BN, CH)
    y = jnp.dot(xb, w_ref[...], preferred_element_type=jnp.float32)
    yb = y.astype(jnp.bfloat16)
    for s in range(SEQ):
        for p in range(BATCH // 2):
            lo = yb[(2 * p) * BN:(2 * p + 1) * BN, s * CH:(s + 1) * CH]
            hi = yb[(2 * p + 1) * BN:(2 * p + 2) * BN, s * CH:(s + 1) * CH]
            lo32 = jax.lax.bitcast_convert_type(
                lo, jnp.uint16).astype(jnp.int32)
            hi32 = jax.lax.bitcast_convert_type(
                hi, jnp.uint16).astype(jnp.int32)
            y_ref[s * 2 + p] = jnp.bitwise_or(
                lo32, jnp.left_shift(hi32, 16))


def _stage1(x_bf, w_all):
    return pl.pallas_call(
        _mm_body,
        grid=(N_NODES // BN,),
        in_specs=[
            pl.BlockSpec((BATCH, BN, CH), lambda i: (0, i, 0)),
            pl.BlockSpec((CH, SEQ * CH), lambda i: (0, 0)),
        ],
        out_specs=pl.BlockSpec((SEQ * 2, BN, CH), lambda i: (0, i, 0)),
        out_shape=jax.ShapeDtypeStruct((SEQ * 2, N_NODES, CH), jnp.int32),
    )(x_bf, w_all)


CN = 16
NCHUNK = N_NODES // CN
NWORK = 32
KPW = (NCHUNK + NWORK - 1) // NWORK
NROW = SEQ * 2 * CN
IDXW = NROW
_STREAMS = ((0, 128), (128, 128), (256, 32))

@functools.lru_cache(maxsize=None)
def _mesh():
    return plsc.VectorSubcoreMesh(core_axis_name="c", subcore_axis_name="s")


@functools.lru_cache(maxsize=None)
def _stage2_kernel():
    return pl.kernel(
        _stage2_body,
        out_type=jax.ShapeDtypeStruct((BATCH, N_NODES, CH), jnp.float32),
        mesh=_mesh(),
        compiler_params=pltpu.CompilerParams(needs_layout_passes=False),
        scratch_types=[
            pltpu.VMEM((KPW * IDXW,), jnp.int32),
            pltpu.VMEM((2, NROW, CH), jnp.int32),
            pltpu.VMEM((2, BATCH, CN, CH), jnp.float32),
            pltpu.VMEM((CH,), jnp.float32),
            pltpu.SemaphoreType.DMA,
            pltpu.SemaphoreType.DMA,
            pltpu.SemaphoreType.DMA,
            pltpu.SemaphoreType.DMA,
        ],
    )


def _bf16_pair_to_f32(tot):
    return plsc.unpack(tot, format=plsc.PackFormat.INTERLEAVED)


def _elu(v):
    return jnp.where(v > 0, v, jnp.exp(v) - 1.0)


def _stage2_body(y_hbm, idxc_hbm, bias_hbm, o_hbm, idx_v, buf_v,
                 res_v, bias_v, sem0, sem1, osem0, osem1):
    wid = lax.axis_index("s") * 2 + lax.axis_index("c")
    sems = (sem0, sem1)
    osems = (osem0, osem1)
    pltpu.sync_copy(bias_hbm, bias_v)
    pltpu.sync_copy(idxc_hbm.at[pl.ds(wid * (KPW * IDXW), KPW * IDXW)], idx_v)

    def fire(j, par):
        for off, n in _STREAMS:
            pltpu.async_copy(
                y_hbm.at[idx_v.at[pl.ds(j * IDXW + off, n)]],
                buf_v.at[par, pl.ds(off, n)], sems[par])

    def drain(j, par):
        for off, n in _STREAMS:
            pltpu.make_async_copy(
                y_hbm.at[idx_v.at[pl.ds(j * IDXW + off, n)]],
                buf_v.at[par, pl.ds(off, n)], sems[par]).wait()

    def drain_out(par):
        for b in range(BATCH):
            pltpu.make_async_copy(
                res_v.at[par, b], o_hbm.at[b, pl.ds(0, CN)],
                osems[par]).wait()

    def consume(ck, j, par):
        drain(j, par)

        @pl.when(j >= 2)
        def _():
            drain_out(par)

        def row_body(r, c):
            for p in range(2):
                for g in range(CH // 16):
                    sl16 = pl.ds(g * 16, 16)
                    terms = [
                        plsc.bitcast(
                            buf_v[par, (s * 2 + p) * CN + r, sl16],
                            jnp.bfloat16)
                        for s in range(SEQ)
                    ]
                    while len(terms) > 1:
                        nxt = []
                        for t in range(0, len(terms) - 1, 2):
                            nxt.append(terms[t] + terms[t + 1])
                        if len(terms) % 2:
                            nxt.append(terms[-1])
                        terms = nxt
                    lo, hi = _bf16_pair_to_f32(terms[0])
                    bias = bias_v[sl16]
                    res_v[par, 2 * p, r, sl16] = _elu(lo + bias)
                    res_v[par, 2 * p + 1, r, sl16] = _elu(hi + bias)
            return c

        lax.fori_loop(0, CN, row_body, 0)
        base = ck * CN
        for b in range(BATCH):
            pltpu.async_copy(
                res_v.at[par, b], o_hbm.at[b, pl.ds(base, CN)], osems[par])

    fire(0, 0)

    def pair_body(jj, carry):
        for par in range(2):
            j = 2 * jj + par
            ck = wid + j * NWORK
            nck = ck + NWORK

            @pl.when(nck < NCHUNK)
            def _():
                fire(j + 1, 1 - par)

            @pl.when(ck < NCHUNK)
            def _():
                consume(ck, j, par)

        return carry

    lax.fori_loop(0, (KPW + 1) // 2, pair_body, 0)
    drain_out(0)
    drain_out(1)


EC = 128
NEC = (NNZ + EC - 1) // EC
EP = NEC * EC
MR = 24
M_PAD = 12544
RZ = 128
NRC = M_PAD // RZ


@functools.lru_cache(maxsize=None)
def _stage3_kernel():
    return pl.kernel(
        _stage3_body,
        out_type=jax.ShapeDtypeStruct((BATCH * M_PAD, CH), jnp.float32),
        mesh=_mesh(),
        compiler_params=pltpu.CompilerParams(needs_layout_passes=False),
        scratch_types=[
            pltpu.VMEM((2, MR, CH), jnp.int32),
            pltpu.VMEM((EC, CH), jnp.float32),
            pltpu.VMEM_SHARED((M_PAD, CH), jnp.float32),
            pltpu.SemaphoreType.DMA,
            pltpu.SemaphoreType.DMA,
            pltpu.SemaphoreType.DMA,
        ],
    )


def _stage3_body(o_hbm, emeta_hbm, p_hbm, ebuf_v, g_v, shared,
                 semm0, semm1, sem):
    cid = lax.axis_index("c")
    sid = lax.axis_index("s")
    semms = (semm0, semm1)
    nech = (NEC + 15) // 16

    def meta_fire(i, par):
        ck = sid + i * 16
        pltpu.async_copy(emeta_hbm.at[pl.ds(ck * MR, MR)],
                         ebuf_v.at[par], semms[par])

    def meta_wait(i, par):
        ck = sid + i * 16
        pltpu.make_async_copy(emeta_hbm.at[pl.ds(ck * MR, MR)],
                              ebuf_v.at[par], semms[par]).wait()

    for bp in range(2):
        b = bp * 2 + cid

        def zb(r, c):
            for j in range(CH // 16):
                g_v[r, pl.ds(j * 16, 16)] = jnp.zeros((16,), jnp.float32)
            return c

        lax.fori_loop(0, RZ, zb, 0)

        def zchunk(i, c):
            ck = sid + i * 16

            @pl.when(ck < NRC)
            def _():
                pltpu.async_copy(g_v, shared.at[pl.ds(ck * RZ, RZ)], sem)

            return c

        lax.fori_loop(0, (NRC + 15) // 16, zchunk, 0)

        def zdrain(i, c):
            ck = sid + i * 16

            @pl.when(ck < NRC)
            def _():
                pltpu.make_async_copy(
                    g_v, shared.at[pl.ds(ck * RZ, RZ)], sem).wait()

            return c

        lax.fori_loop(0, (NRC + 15) // 16, zdrain, 0)
        plsc.subcore_barrier()

        meta_fire(0, 0)

        def epair(ii, c):
            for par in range(2):
                i = 2 * ii + par
                ck = sid + i * 16

                @pl.when(sid + (i + 1) * 16 < NEC)
                def _():
                    meta_fire(i + 1, 1 - par)

                @pl.when(ck < NEC)
                def _():
                    meta_wait(i, par)
                    off = b * N_NODES
                    for j in range(CH // 16):
                        sl = pl.ds(j * 16, 16)
                        ebuf_v[par, 0, sl] = ebuf_v[par, 0, sl] + off
                    pltpu.async_copy(
                        o_hbm.at[ebuf_v.at[par, 0]], g_v, sem).wait()

                    def scale(r, cc):
                        for cg in range(8):
                            vk = plsc.bitcast(
                                ebuf_v[par, 2 + r, pl.ds(cg * 16, 16)],
                                jnp.float32)
                            e = r * 8 + cg
                            for j in range(CH // 16):
                                sl = (e, pl.ds(j * 16, 16))
                                g_v[sl] = g_v[sl] * vk
                        return cc

                    lax.fori_loop(0, 16, scale, 0)
                    pltpu.sync_copy(g_v, shared.at[ebuf_v.at[par, 1]],
                                    add=True)

            return c

        lax.fori_loop(0, (nech + 1) // 2, epair, 0)
        plsc.subcore_barrier()

        def wchunk(i, c):
            ck = sid + i * 16

            @pl.when(ck < NRC)
            def _():
                r0 = ck * RZ
                pltpu.async_copy(shared.at[pl.ds(r0, RZ)],
                                 p_hbm.at[pl.ds(b * M_PAD + r0, RZ)], sem)

            return c

        lax.fori_loop(0, (NRC + 15) // 16, wchunk, 0)

        def wdrain(i, c):
            ck = sid + i * 16

            @pl.when(ck < NRC)
            def _():
                r0 = ck * RZ
                pltpu.make_async_copy(
                    shared.at[pl.ds(r0, RZ)],
                    p_hbm.at[pl.ds(b * M_PAD + r0, RZ)], sem).wait()

            return c

        lax.fori_loop(0, (NRC + 15) // 16, wdrain, 0)
        plsc.subcore_barrier()


def kernel(x, W, b, down_value, indices, down_row, down_col):
    w_all = W.reshape(SEQ, CH, CH).transpose(1, 0, 2).reshape(CH, SEQ * CH)
    y = _stage1(x.astype(jnp.bfloat16), w_all.astype(jnp.bfloat16))

    idxt = indices.astype(jnp.int32).T.reshape(SEQ, NCHUNK, CN)
    offs = ((jnp.arange(SEQ) * 2)[:, None, None, None]
            + jnp.arange(2)[None, None, :, None]) * N_NODES
    gidx = idxt[:, :, None, :] + offs
    idxc = gidx.transpose(1, 0, 2, 3).reshape(NCHUNK, IDXW)
    idxc = jnp.pad(idxc, ((0, KPW * NWORK - NCHUNK), (0, 0)))
    idxc = idxc.reshape(KPW, NWORK, IDXW).transpose(1, 0, 2).reshape(-1)
    o = _stage2_kernel()(y.reshape(SEQ * 2 * N_NODES, CH), idxc, b)

    pad = EP - NNZ
    colp = jnp.pad(down_col.astype(jnp.int32), (0, pad)).reshape(NEC, 1, CH)
    rowp = jnp.pad(down_row.astype(jnp.int32), (0, pad)).reshape(NEC, 1, CH)
    val16 = jax.lax.bitcast_convert_type(
        jnp.repeat(jnp.pad(down_value, (0, pad)), 16),
        jnp.int32).reshape(NEC, 16, CH)
    zpad = jnp.zeros((NEC, MR - 18, CH), jnp.int32)
    emeta = jnp.concatenate([colp, rowp, val16, zpad], axis=1)
    o2 = o.reshape(BATCH * N_NODES, CH)
    p = _stage3_kernel()(o2, emeta.reshape(NEC * MR, CH))
    return p.reshape(BATCH, M_PAD, CH)[:, :M_NODES]
